# Initial kernel scaffold; baseline (speedup 1.0000x reference)
import jax
import jax.numpy as jnp
from jax.experimental import pallas as pl


def kernel(token_ids, weight):
    raise NotImplementedError("write your pallas kernel here")


# speedup vs baseline: 1.8767x; 1.8767x over previous
import functools

import jax
import jax.numpy as jnp
from jax import lax
from jax.experimental import pallas as pl
from jax.experimental.pallas import tpu as pltpu
from jax.experimental.pallas import tpu_sc as plsc

NUM_EMBEDDINGS = 1000000
EMBEDDING_DIM = 64
BATCH = 16384
SEQ = 50

_NC = 2
_NS = 16
_NW = _NC * _NS

_B_TOTAL = BATCH * SEQ
_CHUNK = 128
_NBUF = 4
_B_PER_W = _B_TOTAL // _NW
_N_CHUNKS = _B_PER_W // _CHUNK
_N_ROUNDS = _N_CHUNKS // _NBUF

assert _B_PER_W * _NW == _B_TOTAL
assert _CHUNK * _N_CHUNKS == _B_PER_W
assert _NBUF * _N_ROUNDS == _N_CHUNKS


def _body(ids_hbm, table_hbm, out_hbm, idx_v, rows_v, *sems):
    gsem = sems[:_NBUF]
    osem = sems[_NBUF:]
    wid = lax.axis_index("s") * _NC + lax.axis_index("c")
    base = wid * _B_PER_W

    pltpu.sync_copy(ids_hbm.at[wid], idx_v)

    for b in range(_NBUF):
        pltpu.async_copy(table_hbm.at[idx_v.at[b]], rows_v.at[b], gsem[b])

    def round_body(r, carry):
        for b in range(_NBUF):
            c = r * _NBUF + b
            dst = out_hbm.at[pl.ds(base + c * _CHUNK, _CHUNK)]
            pltpu.make_async_copy(
                table_hbm.at[idx_v.at[c]], rows_v.at[b], gsem[b]).wait()
            pltpu.async_copy(rows_v.at[b], dst, osem[b])
            pltpu.make_async_copy(rows_v.at[b], dst, osem[b]).wait()
            nxt = c + _NBUF

            @pl.when(nxt < _N_CHUNKS)
            def _():
                pltpu.async_copy(
                    table_hbm.at[idx_v.at[nxt]], rows_v.at[b], gsem[b])

        return carry

    lax.fori_loop(0, _N_ROUNDS, round_body, 0)


@jax.jit
def kernel(token_ids, weight):
    ids = token_ids.reshape(_NW, _N_CHUNKS, _CHUNK).astype(jnp.int32)
    run = pl.kernel(
        _body,
        out_type=jax.ShapeDtypeStruct((_B_TOTAL, EMBEDDING_DIM), jnp.float32),
        mesh=plsc.VectorSubcoreMesh(
            core_axis_name="c", subcore_axis_name="s",
            num_cores=_NC, num_subcores=_NS),
        compiler_params=pltpu.CompilerParams(use_tc_tiling_on_sc=False),
        scratch_types=[
            pltpu.VMEM((_N_CHUNKS, _CHUNK), jnp.int32),
            pltpu.VMEM((_NBUF, _CHUNK, EMBEDDING_DIM), jnp.float32),
        ] + [pltpu.SemaphoreType.DMA] * (2 * _NBUF),
    )
    out = run(ids, weight)
    return out.reshape(BATCH, SEQ, EMBEDDING_DIM)
